# Initial kernel scaffold; baseline (speedup 1.0000x reference)
import jax
import jax.numpy as jnp
from jax.experimental import pallas as pl


def kernel(node_feats, edge_index, graph_ids, W1a, b1a, W1b, b1b, W2a, b2a, W2b, b2b, Wp0, bp0, Wp1, bp1, Wp2, bp2, Wm1, bm1, Wm2, bm2, Wmean, bmean, Wstd, bstd):
    raise NotImplementedError("write your pallas kernel here")


# speedup vs baseline: 7.9305x; 7.9305x over previous
import functools

import jax
import jax.numpy as jnp
from jax import lax
from jax.experimental import pallas as pl
from jax.experimental.pallas import tpu as pltpu
from jax.experimental.pallas import tpu_sc as plsc

N = 100000
E = 1600000
G = 16
NP = 100352
NT = NP // 16
EP = 1600512
EROWS = EP // 128
S = 8
CHUNKS = EROWS // S


def _make_agg(num_groups):
    npasses = num_groups // 2
    mesh = plsc.VectorSubcoreMesh(core_axis_name="c", subcore_axis_name="s")

    @functools.partial(
        pl.kernel,
        out_type=jax.ShapeDtypeStruct((num_groups, NP, 16), jnp.float32),
        mesh=mesh,
        scratch_types=[
            pltpu.VMEM((S, 128), jnp.int32),
            pltpu.VMEM((S, 128), jnp.int32),
            pltpu.VMEM((S * 128, 16), jnp.float32),
            pltpu.VMEM_SHARED((NP, 16), jnp.float32),
            pltpu.SemaphoreType.DMA,
            pltpu.SemaphoreType.DMA,
        ],
        compiler_params=pltpu.CompilerParams(use_tc_tiling_on_sc=False),
    )
    def agg(*refs):
        tables = refs[:num_groups]
        srcg, dstg, zeros_hbm = refs[num_groups:num_groups + 3]
        out = refs[num_groups + 3]
        src_idx, dst_idx, rows, acc, gsem, ssem = refs[num_groups + 4:]

        c = lax.axis_index("c")
        s = lax.axis_index("s")
        lo = s * NT
        cnt = 97 + (s < 11).astype(jnp.int32)

        def one_pass(group):
            table = tables[group]
            pltpu.sync_copy(zeros_hbm.at[pl.ds(lo, NT)], acc.at[pl.ds(lo, NT)])
            plsc.subcore_barrier()

            def chunk(i, carry):
                row0 = (s + i * 16) * S
                pltpu.sync_copy(srcg.at[pl.ds(row0, S)], src_idx)
                pltpu.sync_copy(dstg.at[pl.ds(row0, S)], dst_idx)
                hs = [
                    pltpu.async_copy(table.at[src_idx.at[j]],
                                     rows.at[pl.ds(j * 128, 128)], gsem)
                    for j in range(S)
                ]
                for h in hs:
                    h.wait()
                ws = [
                    pltpu.async_copy(rows.at[pl.ds(j * 128, 128)],
                                     acc.at[dst_idx.at[j]], ssem, add=True)
                    for j in range(S)
                ]
                for w in ws:
                    w.wait()
                return carry

            lax.fori_loop(0, cnt, chunk, 0)
            plsc.subcore_barrier()
            pltpu.sync_copy(acc.at[pl.ds(lo, NT)],
                            out.at[group, pl.ds(lo, NT)])

        def run(groups):
            for g in groups:
                one_pass(g)

        pl.when(c == 0)(lambda: run(range(npasses)))
        pl.when(c == 1)(lambda: run(range(npasses, num_groups)))

    return agg


@functools.cache
def _agg(num_groups):
    return _make_agg(num_groups)


RB = 4000


def _mlp1_body(h0a, h0b, agg0a, agg0b, oh, W1a, b1a, W1b, b1b,
               o0, o1, o2, o3, p0, p1):
    h0 = jnp.concatenate([h0a[...], h0b[...]], axis=1)
    x = jnp.concatenate([h0a[...] + agg0a[0], h0b[...] + agg0b[0]], axis=1)
    t = jnp.maximum(jnp.dot(x, W1a[...], preferred_element_type=jnp.float32)
                    + b1a[...], 0.0)
    h1 = jnp.maximum(jnp.dot(t, W1b[...], preferred_element_type=jnp.float32)
                     + b1b[...], 0.0)
    o0[...] = h1[:, 0:16]
    o1[...] = h1[:, 16:32]
    o2[...] = h1[:, 32:48]
    o3[...] = h1[:, 48:64]
    ohT = oh[...]
    pp0 = lax.dot_general(ohT, h0, (((0,), (0,)), ((), ())),
                          preferred_element_type=jnp.float32)
    pp1 = lax.dot_general(ohT, h1, (((0,), (0,)), ((), ())),
                          preferred_element_type=jnp.float32)

    @pl.when(pl.program_id(0) == 0)
    def _():
        p0[...] = pp0
        p1[...] = pp1

    @pl.when(pl.program_id(0) != 0)
    def _():
        p0[...] += pp0
        p1[...] += pp1


def _mlp1(h0a, h0b, agg0, oh, W1a, b1a, W1b, b1b):
    grid = (N // RB,)
    return pl.pallas_call(
        _mlp1_body,
        grid=grid,
        in_specs=[
            pl.BlockSpec((RB, 16), lambda i: (i, 0)),
            pl.BlockSpec((RB, 16), lambda i: (i, 0)),
            pl.BlockSpec((1, RB, 16), lambda i: (0, i, 0)),
            pl.BlockSpec((1, RB, 16), lambda i: (1, i, 0)),
            pl.BlockSpec((RB, G), lambda i: (i, 0)),
            pl.BlockSpec((32, 64), lambda i: (0, 0)),
            pl.BlockSpec((1, 64), lambda i: (0, 0)),
            pl.BlockSpec((64, 64), lambda i: (0, 0)),
            pl.BlockSpec((1, 64), lambda i: (0, 0)),
        ],
        out_specs=[
            pl.BlockSpec((RB, 16), lambda i: (i, 0)),
            pl.BlockSpec((RB, 16), lambda i: (i, 0)),
            pl.BlockSpec((RB, 16), lambda i: (i, 0)),
            pl.BlockSpec((RB, 16), lambda i: (i, 0)),
            pl.BlockSpec((G, 32), lambda i: (0, 0)),
            pl.BlockSpec((G, 64), lambda i: (0, 0)),
        ],
        out_shape=[
            jax.ShapeDtypeStruct((N, 16), jnp.float32),
            jax.ShapeDtypeStruct((N, 16), jnp.float32),
            jax.ShapeDtypeStruct((N, 16), jnp.float32),
            jax.ShapeDtypeStruct((N, 16), jnp.float32),
            jax.ShapeDtypeStruct((G, 32), jnp.float32),
            jax.ShapeDtypeStruct((G, 64), jnp.float32),
        ],
    )(h0a, h0b, agg0, agg0, oh, W1a, b1a, W1b, b1b)


def _mlp2_body(h1a, h1b, h1c, h1d, a0, a1, a2, a3, oh,
               W2a, b2a, W2b, b2b, p2):
    hs = [h1a, h1b, h1c, h1d]
    ags = [a0, a1, a2, a3]
    x = jnp.concatenate([hs[k][...] + ags[k][0] for k in range(4)], axis=1)
    t = jnp.maximum(jnp.dot(x, W2a[...], preferred_element_type=jnp.float32)
                    + b2a[...], 0.0)
    h2 = jnp.maximum(jnp.dot(t, W2b[...], preferred_element_type=jnp.float32)
                     + b2b[...], 0.0)
    pp2 = lax.dot_general(oh[...], h2, (((0,), (0,)), ((), ())),
                          preferred_element_type=jnp.float32)

    @pl.when(pl.program_id(0) == 0)
    def _():
        p2[...] = pp2

    @pl.when(pl.program_id(0) != 0)
    def _():
        p2[...] += pp2


def _mlp2(h1a, h1b, h1c, h1d, agg1, oh, W2a, b2a, W2b, b2b):
    grid = (N // RB,)

    def agg_spec(k):
        return pl.BlockSpec((1, RB, 16), lambda i, k=k: (k, i, 0))

    return pl.pallas_call(
        _mlp2_body,
        grid=grid,
        in_specs=[
            pl.BlockSpec((RB, 16), lambda i: (i, 0)),
            pl.BlockSpec((RB, 16), lambda i: (i, 0)),
            pl.BlockSpec((RB, 16), lambda i: (i, 0)),
            pl.BlockSpec((RB, 16), lambda i: (i, 0)),
            agg_spec(0), agg_spec(1), agg_spec(2), agg_spec(3),
            pl.BlockSpec((RB, G), lambda i: (i, 0)),
            pl.BlockSpec((64, 64), lambda i: (0, 0)),
            pl.BlockSpec((1, 64), lambda i: (0, 0)),
            pl.BlockSpec((64, 64), lambda i: (0, 0)),
            pl.BlockSpec((1, 64), lambda i: (0, 0)),
        ],
        out_specs=pl.BlockSpec((G, 64), lambda i: (0, 0)),
        out_shape=jax.ShapeDtypeStruct((G, 64), jnp.float32),
    )(h1a, h1b, h1c, h1d, agg1, agg1, agg1, agg1, oh, W2a, b2a, W2b, b2b)


def _heads_body(p0, p1, p2, Wp0, bp0, Wp1, bp1, Wp2, bp2,
                Wm1, bm1, Wm2, bm2, Wmean, bmean, Wstd, bstd, mean, std):
    score = (jnp.dot(p0[...], Wp0[...], preferred_element_type=jnp.float32)
             + bp0[...]
             + jnp.dot(p1[...], Wp1[...], preferred_element_type=jnp.float32)
             + bp1[...]
             + jnp.dot(p2[...], Wp2[...], preferred_element_type=jnp.float32)
             + bp2[...])
    f = jnp.maximum(jnp.dot(score, Wm1[...], preferred_element_type=jnp.float32)
                    + bm1[...], 0.0)
    f = jnp.maximum(jnp.dot(f, Wm2[...], preferred_element_type=jnp.float32)
                    + bm2[...], 0.0)
    mean[...] = jnp.dot(f, Wmean[...], preferred_element_type=jnp.float32) \
        + bmean[...]
    z = jnp.dot(f, Wstd[...], preferred_element_type=jnp.float32) + bstd[...]
    std[...] = jnp.maximum(z, 0.0) + jnp.log1p(jnp.exp(-jnp.abs(z)))


def _heads(p0, p1, p2, Wp0, bp0, Wp1, bp1, Wp2, bp2,
           Wm1, bm1, Wm2, bm2, Wmean, bmean, Wstd, bstd):
    return pl.pallas_call(
        _heads_body,
        out_shape=[
            jax.ShapeDtypeStruct((G, 32), jnp.float32),
            jax.ShapeDtypeStruct((G, 32), jnp.float32),
        ],
    )(p0, p1, p2, Wp0, bp0, Wp1, bp1, Wp2, bp2,
      Wm1, bm1, Wm2, bm2, Wmean, bmean, Wstd, bstd)


def kernel(node_feats, edge_index, graph_ids,
           W1a, b1a, W1b, b1b, W2a, b2a, W2b, b2b,
           Wp0, bp0, Wp1, bp1, Wp2, bp2,
           Wm1, bm1, Wm2, bm2, Wmean, bmean, Wstd, bstd):
    h0a = node_feats[:, 0:16]
    h0b = node_feats[:, 16:32]
    npad = EP - E
    src = jnp.concatenate(
        [edge_index[0], jnp.zeros((npad,), jnp.int32)]).reshape(EROWS, 128)
    dst = jnp.concatenate(
        [edge_index[1],
         N + (jnp.arange(npad, dtype=jnp.int32) % (NP - N))]) \
        .reshape(EROWS, 128)
    zeros = jnp.zeros((NP, 16), jnp.float32)
    oh = (graph_ids[:, None] == jnp.arange(G, dtype=jnp.int32)[None, :]) \
        .astype(jnp.float32)

    agg0 = _agg(2)(h0a, h0b, src, dst, zeros)
    h1a, h1b, h1c, h1d, p0, p1 = _mlp1(
        h0a, h0b, agg0, oh, W1a, b1a.reshape(1, -1), W1b, b1b.reshape(1, -1))
    agg1 = _agg(4)(h1a, h1b, h1c, h1d, src, dst, zeros)
    p2 = _mlp2(h1a, h1b, h1c, h1d, agg1, oh,
               W2a, b2a.reshape(1, -1), W2b, b2b.reshape(1, -1))
    mean, std = _heads(
        p0, p1, p2, Wp0, bp0.reshape(1, -1), Wp1, bp1.reshape(1, -1),
        Wp2, bp2.reshape(1, -1), Wm1, bm1.reshape(1, -1),
        Wm2, bm2.reshape(1, -1), Wmean, bmean.reshape(1, -1),
        Wstd, bstd.reshape(1, -1))
    return mean, std


# speedup vs baseline: 9.0428x; 1.1403x over previous
import functools

import jax
import jax.numpy as jnp
from jax import lax
from jax.experimental import pallas as pl
from jax.experimental.pallas import tpu as pltpu
from jax.experimental.pallas import tpu_sc as plsc

N = 100000
E = 1600000
G = 16
NP = 100352
NT = NP // 16
EP = 1600512
EROWS = EP // 128
S = 8
CHUNKS = EROWS // S


def _make_agg(num_groups, table_rows):
    npasses = num_groups // 2
    mesh = plsc.VectorSubcoreMesh(core_axis_name="c", subcore_axis_name="s")

    @functools.partial(
        pl.kernel,
        out_type=jax.ShapeDtypeStruct((NP, num_groups, 16), jnp.float32),
        mesh=mesh,
        scratch_types=[
            pltpu.VMEM((S, 128), jnp.int32),
            pltpu.VMEM((S, 128), jnp.int32),
            pltpu.VMEM((S * 128, 16), jnp.float32),
            pltpu.VMEM_SHARED((NP, 16), jnp.float32),
            pltpu.SemaphoreType.DMA,
            pltpu.SemaphoreType.DMA,
        ],
        compiler_params=pltpu.CompilerParams(use_tc_tiling_on_sc=False),
    )
    def agg(*refs):
        table = refs[0]
        idxs = refs[1:1 + num_groups]
        dstg, zeros_hbm = refs[1 + num_groups:3 + num_groups]
        out = refs[3 + num_groups]
        src_idx, dst_idx, rows, acc, gsem, ssem = refs[4 + num_groups:]

        c = lax.axis_index("c")
        s = lax.axis_index("s")
        lo = s * NT
        cnt = 97 + (s < 11).astype(jnp.int32)

        def one_pass(group):
            idxg = idxs[group]
            pltpu.sync_copy(zeros_hbm.at[pl.ds(lo, NT)], acc.at[pl.ds(lo, NT)])
            plsc.subcore_barrier()

            def chunk(i, carry):
                row0 = (s + i * 16) * S
                pltpu.sync_copy(idxg.at[pl.ds(row0, S)], src_idx)
                pltpu.sync_copy(dstg.at[pl.ds(row0, S)], dst_idx)
                hs = [
                    pltpu.async_copy(table.at[src_idx.at[j]],
                                     rows.at[pl.ds(j * 128, 128)], gsem)
                    for j in range(S)
                ]
                for h in hs:
                    h.wait()
                ws = [
                    pltpu.async_copy(rows.at[pl.ds(j * 128, 128)],
                                     acc.at[dst_idx.at[j]], ssem, add=True)
                    for j in range(S)
                ]
                for w in ws:
                    w.wait()
                return carry

            lax.fori_loop(0, cnt, chunk, 0)
            plsc.subcore_barrier()
            pltpu.sync_copy(acc.at[pl.ds(lo, NT)],
                            out.at[pl.ds(lo, NT), group])

        def run(groups):
            for g in groups:
                one_pass(g)

        pl.when(c == 0)(lambda: run(range(npasses)))
        pl.when(c == 1)(lambda: run(range(npasses, num_groups)))

    return agg


@functools.cache
def _agg(num_groups, table_rows):
    return _make_agg(num_groups, table_rows)


RB = 4000


def _unpack(xp, w):
    m = xp.shape[0]
    k = 128 // w
    pieces = [xp[:, w * j:w * (j + 1)].reshape(m, 1, w) for j in range(k)]
    return jnp.concatenate(pieces, axis=1).reshape(m * k, w)


def _pack(x):
    n, w = x.shape
    k = 128 // w
    x3 = x.reshape(n // k, k, w)
    return jnp.concatenate([x3[:, j, :] for j in range(k)], axis=1)


def _mlp1_body(nf, agg0, oh, W1a, b1a, W1b, b1b, h1p, p0, p1):
    h0 = nf[:, 0:32]
    x = h0 + _unpack(agg0[...], 32)
    t = jnp.maximum(jnp.dot(x, W1a[...], preferred_element_type=jnp.float32)
                    + b1a[...], 0.0)
    h1 = jnp.maximum(jnp.dot(t, W1b[...], preferred_element_type=jnp.float32)
                     + b1b[...], 0.0)
    h1p[...] = _pack(h1)
    ohb = _unpack(oh[0], G)
    pp0 = lax.dot_general(ohb, h0, (((0,), (0,)), ((), ())),
                          preferred_element_type=jnp.float32)
    pp1 = lax.dot_general(ohb, h1, (((0,), (0,)), ((), ())),
                          preferred_element_type=jnp.float32)

    @pl.when(pl.program_id(0) == 0)
    def _():
        p0[...] = pp0
        p1[...] = pp1

    @pl.when(pl.program_id(0) != 0)
    def _():
        p0[...] += pp0
        p1[...] += pp1


def _mlp1(nf, agg0v, ohp, W1a, b1a, W1b, b1b):
    grid = (N // RB,)
    return pl.pallas_call(
        _mlp1_body,
        grid=grid,
        in_specs=[
            pl.BlockSpec((RB, 128), lambda i: (i, 0)),
            pl.BlockSpec((RB // 4, 128), lambda i: (i, 0)),
            pl.BlockSpec((1, RB // 8, 128), lambda i: (i, 0, 0)),
            pl.BlockSpec((32, 64), lambda i: (0, 0)),
            pl.BlockSpec((1, 64), lambda i: (0, 0)),
            pl.BlockSpec((64, 64), lambda i: (0, 0)),
            pl.BlockSpec((1, 64), lambda i: (0, 0)),
        ],
        out_specs=[
            pl.BlockSpec((RB // 2, 128), lambda i: (i, 0)),
            pl.BlockSpec((G, 32), lambda i: (0, 0)),
            pl.BlockSpec((G, 64), lambda i: (0, 0)),
        ],
        out_shape=[
            jax.ShapeDtypeStruct((N // 2, 128), jnp.float32),
            jax.ShapeDtypeStruct((G, 32), jnp.float32),
            jax.ShapeDtypeStruct((G, 64), jnp.float32),
        ],
    )(nf, agg0v, ohp, W1a, b1a, W1b, b1b)


def _mlp2_body(h1p, agg1, oh, W2a, b2a, W2b, b2b, p2):
    h1 = _unpack(h1p[...], 64)
    x = h1 + _unpack(agg1[...], 64)
    t = jnp.maximum(jnp.dot(x, W2a[...], preferred_element_type=jnp.float32)
                    + b2a[...], 0.0)
    h2 = jnp.maximum(jnp.dot(t, W2b[...], preferred_element_type=jnp.float32)
                     + b2b[...], 0.0)
    pp2 = lax.dot_general(_unpack(oh[0], G), h2, (((0,), (0,)), ((), ())),
                          preferred_element_type=jnp.float32)

    @pl.when(pl.program_id(0) == 0)
    def _():
        p2[...] = pp2

    @pl.when(pl.program_id(0) != 0)
    def _():
        p2[...] += pp2


def _mlp2(h1p, agg1v, ohp, W2a, b2a, W2b, b2b):
    grid = (N // RB,)
    return pl.pallas_call(
        _mlp2_body,
        grid=grid,
        in_specs=[
            pl.BlockSpec((RB // 2, 128), lambda i: (i, 0)),
            pl.BlockSpec((RB // 2, 128), lambda i: (i, 0)),
            pl.BlockSpec((1, RB // 8, 128), lambda i: (i, 0, 0)),
            pl.BlockSpec((64, 64), lambda i: (0, 0)),
            pl.BlockSpec((1, 64), lambda i: (0, 0)),
            pl.BlockSpec((64, 64), lambda i: (0, 0)),
            pl.BlockSpec((1, 64), lambda i: (0, 0)),
        ],
        out_specs=pl.BlockSpec((G, 64), lambda i: (0, 0)),
        out_shape=jax.ShapeDtypeStruct((G, 64), jnp.float32),
    )(h1p, agg1v, ohp, W2a, b2a, W2b, b2b)


def _heads_body(p0, p1, p2, Wp0, bp0, Wp1, bp1, Wp2, bp2,
                Wm1, bm1, Wm2, bm2, Wmean, bmean, Wstd, bstd, mean, std):
    score = (jnp.dot(p0[...], Wp0[...], preferred_element_type=jnp.float32)
             + bp0[...]
             + jnp.dot(p1[...], Wp1[...], preferred_element_type=jnp.float32)
             + bp1[...]
             + jnp.dot(p2[...], Wp2[...], preferred_element_type=jnp.float32)
             + bp2[...])
    f = jnp.maximum(jnp.dot(score, Wm1[...], preferred_element_type=jnp.float32)
                    + bm1[...], 0.0)
    f = jnp.maximum(jnp.dot(f, Wm2[...], preferred_element_type=jnp.float32)
                    + bm2[...], 0.0)
    mean[...] = jnp.dot(f, Wmean[...], preferred_element_type=jnp.float32) \
        + bmean[...]
    z = jnp.dot(f, Wstd[...], preferred_element_type=jnp.float32) + bstd[...]
    std[...] = jnp.maximum(z, 0.0) + jnp.log1p(jnp.exp(-jnp.abs(z)))


def _heads(p0, p1, p2, Wp0, bp0, Wp1, bp1, Wp2, bp2,
           Wm1, bm1, Wm2, bm2, Wmean, bmean, Wstd, bstd):
    return pl.pallas_call(
        _heads_body,
        out_shape=[
            jax.ShapeDtypeStruct((G, 32), jnp.float32),
            jax.ShapeDtypeStruct((G, 32), jnp.float32),
        ],
    )(p0, p1, p2, Wp0, bp0, Wp1, bp1, Wp2, bp2,
      Wm1, bm1, Wm2, bm2, Wmean, bmean, Wstd, bstd)


def kernel(node_feats, edge_index, graph_ids,
           W1a, b1a, W1b, b1b, W2a, b2a, W2b, b2b,
           Wp0, bp0, Wp1, bp1, Wp2, bp2,
           Wm1, bm1, Wm2, bm2, Wmean, bmean, Wstd, bstd):
    npad = EP - E
    srcp = jnp.concatenate([edge_index[0], jnp.zeros((npad,), jnp.int32)])
    dst = jnp.concatenate(
        [edge_index[1],
         N + (jnp.arange(npad, dtype=jnp.int32) % (NP - N))]) \
        .reshape(EROWS, 128)
    zeros = jnp.zeros((NP // 8, 128), jnp.float32).reshape(NP, 16)
    ohp = (graph_ids[:, None] == jnp.arange(G, dtype=jnp.int32)[None, :]) \
        .astype(jnp.float32).reshape(N // RB, RB // 8, 128)

    t0 = node_feats.reshape(8 * N, 16)
    i0 = [(8 * srcp + k).reshape(EROWS, 128) for k in range(2)]
    agg0 = _agg(2, 8 * N)(t0, i0[0], i0[1], dst, zeros)
    agg0v = agg0.reshape(NP // 4, 128)

    h1p, p0, p1 = _mlp1(node_feats, agg0v, ohp,
                        W1a, b1a.reshape(1, -1), W1b, b1b.reshape(1, -1))

    t1 = h1p.reshape(4 * N, 16)
    i1 = [(4 * srcp + k).reshape(EROWS, 128) for k in range(4)]
    agg1 = _agg(4, 4 * N)(t1, i1[0], i1[1], i1[2], i1[3], dst, zeros)
    agg1v = agg1.reshape(NP // 2, 128)

    p2 = _mlp2(h1p, agg1v, ohp,
               W2a, b2a.reshape(1, -1), W2b, b2b.reshape(1, -1))
    mean, std = _heads(
        p0, p1, p2, Wp0, bp0.reshape(1, -1), Wp1, bp1.reshape(1, -1),
        Wp2, bp2.reshape(1, -1), Wm1, bm1.reshape(1, -1),
        Wm2, bm2.reshape(1, -1), Wmean, bmean.reshape(1, -1),
        Wstd, bstd.reshape(1, -1))
    return mean, std


# speedup vs baseline: 14.5432x; 1.6083x over previous
import functools

import jax
import jax.numpy as jnp
from jax import lax
from jax.experimental import pallas as pl
from jax.experimental.pallas import tpu as pltpu
from jax.experimental.pallas import tpu_sc as plsc

N = 100000
E = 1600000
G = 16
NP = 100352
NT = NP // 16
EP = 1638400
EROWS = EP // 128
RT = EROWS // 16
NSB = 20
SBROWS = RT // NSB
TRIPS = SBROWS // 8
CS = 4


def _make_agg(num_groups, table_rows):
    npasses = num_groups // 2
    mesh = plsc.VectorSubcoreMesh(core_axis_name="c", subcore_axis_name="s")

    @functools.partial(
        pl.kernel,
        out_type=jax.ShapeDtypeStruct((NP, num_groups, 16), jnp.float32),
        mesh=mesh,
        scratch_types=[
            pltpu.VMEM((SBROWS, 128), jnp.int32),
            pltpu.VMEM((SBROWS, 128), jnp.int32),
            pltpu.VMEM((CS * 128, 16), jnp.float32),
            pltpu.VMEM((CS * 128, 16), jnp.float32),
            pltpu.VMEM_SHARED((NP, 16), jnp.float32),
            pltpu.SemaphoreType.DMA,
            pltpu.SemaphoreType.DMA,
            pltpu.SemaphoreType.DMA,
            pltpu.SemaphoreType.DMA,
        ],
        compiler_params=pltpu.CompilerParams(use_tc_tiling_on_sc=False),
    )
    def agg(*refs):
        table = refs[0]
        idxs = refs[1:1 + num_groups]
        dstg, zeros_hbm = refs[1 + num_groups:3 + num_groups]
        out = refs[3 + num_groups]
        (isrc, idst, rows_a, rows_b, acc,
         gsem_a, gsem_b, ssem_a, ssem_b) = refs[4 + num_groups:]

        c = lax.axis_index("c")
        s = lax.axis_index("s")
        lo = s * NT

        def drain(sem, rows_buf):
            pltpu.make_async_copy(
                zeros_hbm.at[pl.ds(0, CS * 128)], rows_buf, sem).wait()

        def one_pass(group):
            idxg = idxs[group]
            pltpu.sync_copy(zeros_hbm.at[pl.ds(lo, NT)], acc.at[pl.ds(lo, NT)])
            plsc.subcore_barrier()

            def superblock(sb, carry2):
                @pl.when(sb > 0)
                def _():
                    drain(ssem_a, rows_a)
                    drain(ssem_b, rows_b)
                base_row = s * RT + sb * SBROWS
                pltpu.sync_copy(idxg.at[pl.ds(base_row, SBROWS)], isrc)
                pltpu.sync_copy(dstg.at[pl.ds(base_row, SBROWS)], idst)

                def trip(j, carry):
                    r = j * 2 * CS

                    @pl.when(j > 0)
                    def _():
                        drain(ssem_a, rows_a)
                    ha = [
                        pltpu.async_copy(table.at[isrc.at[r + k]],
                                         rows_a.at[pl.ds(k * 128, 128)],
                                         gsem_a)
                        for k in range(CS)
                    ]

                    @pl.when(j > 0)
                    def _():
                        drain(ssem_b, rows_b)
                    hb = [
                        pltpu.async_copy(table.at[isrc.at[r + CS + k]],
                                         rows_b.at[pl.ds(k * 128, 128)],
                                         gsem_b)
                        for k in range(CS)
                    ]
                    for h in ha:
                        h.wait()
                    for k in range(CS):
                        pltpu.async_copy(rows_a.at[pl.ds(k * 128, 128)],
                                         acc.at[idst.at[r + k]],
                                         ssem_a, add=True)
                    for h in hb:
                        h.wait()
                    for k in range(CS):
                        pltpu.async_copy(rows_b.at[pl.ds(k * 128, 128)],
                                         acc.at[idst.at[r + CS + k]],
                                         ssem_b, add=True)
                    return carry

                lax.fori_loop(0, TRIPS, trip, 0)
                return carry2

            lax.fori_loop(0, NSB, superblock, 0)
            drain(ssem_a, rows_a)
            drain(ssem_b, rows_b)
            plsc.subcore_barrier()
            pltpu.sync_copy(acc.at[pl.ds(lo, NT)],
                            out.at[pl.ds(lo, NT), group])

        def run(groups):
            for g in groups:
                one_pass(g)

        pl.when(c == 0)(lambda: run(range(npasses)))
        pl.when(c == 1)(lambda: run(range(npasses, num_groups)))

    return agg


@functools.cache
def _agg(num_groups, table_rows):
    return _make_agg(num_groups, table_rows)


RB = 4000


def _mlp1_body(h0p, agg0, oh4, W1abd, b1abd, W1bbd, b1bbd, h1p, p0, p1):
    h0 = h0p[...]
    x = h0 + agg0[...]
    t = jnp.maximum(jnp.dot(x, W1abd[...], preferred_element_type=jnp.float32)
                    + b1abd[...], 0.0)
    h1 = jnp.maximum(jnp.dot(t, W1bbd[...], preferred_element_type=jnp.float32)
                     + b1bbd[...], 0.0)
    h1p[:, 0:1, :] = h1[:, 0:128].reshape(RB // 4, 1, 128)
    h1p[:, 1:2, :] = h1[:, 128:256].reshape(RB // 4, 1, 128)
    ohb = oh4[0]
    m0 = lax.dot_general(ohb, h0, (((0,), (0,)), ((), ())),
                         preferred_element_type=jnp.float32)
    m1 = lax.dot_general(ohb, h1, (((0,), (0,)), ((), ())),
                         preferred_element_type=jnp.float32)
    pp0 = sum(m0[16 * j:16 * (j + 1), 32 * j:32 * (j + 1)] for j in range(4))
    pp1 = sum(m1[16 * j:16 * (j + 1), 64 * j:64 * (j + 1)] for j in range(4))

    @pl.when(pl.program_id(0) == 0)
    def _():
        p0[...] = pp0
        p1[...] = pp1

    @pl.when(pl.program_id(0) != 0)
    def _():
        p0[...] += pp0
        p1[...] += pp1


def _mlp1(h0p, agg0v, oh4, W1abd, b1abd, W1bbd, b1bbd):
    grid = (N // RB,)
    return pl.pallas_call(
        _mlp1_body,
        grid=grid,
        in_specs=[
            pl.BlockSpec((RB // 4, 128), lambda i: (i, 0)),
            pl.BlockSpec((RB // 4, 128), lambda i: (i, 0)),
            pl.BlockSpec((1, RB // 4, 64), lambda i: (i, 0, 0)),
            pl.BlockSpec((128, 256), lambda i: (0, 0)),
            pl.BlockSpec((1, 256), lambda i: (0, 0)),
            pl.BlockSpec((256, 256), lambda i: (0, 0)),
            pl.BlockSpec((1, 256), lambda i: (0, 0)),
        ],
        out_specs=[
            pl.BlockSpec((RB // 4, 2, 128), lambda i: (i, 0, 0)),
            pl.BlockSpec((G, 32), lambda i: (0, 0)),
            pl.BlockSpec((G, 64), lambda i: (0, 0)),
        ],
        out_shape=[
            jax.ShapeDtypeStruct((N // 4, 2, 128), jnp.float32),
            jax.ShapeDtypeStruct((G, 32), jnp.float32),
            jax.ShapeDtypeStruct((G, 64), jnp.float32),
        ],
    )(h0p, agg0v, oh4, W1abd, b1abd, W1bbd, b1bbd)


def _mlp2_body(h1pv, agg1, oh2, W2abd, b2abd, W2bbd, b2bbd, p2):
    x = h1pv[...] + agg1[...]
    t = jnp.maximum(jnp.dot(x, W2abd[...], preferred_element_type=jnp.float32)
                    + b2abd[...], 0.0)
    h2 = jnp.maximum(jnp.dot(t, W2bbd[...], preferred_element_type=jnp.float32)
                     + b2bbd[...], 0.0)
    m2 = lax.dot_general(oh2[0], h2, (((0,), (0,)), ((), ())),
                         preferred_element_type=jnp.float32)
    pp2 = sum(m2[16 * j:16 * (j + 1), 64 * j:64 * (j + 1)] for j in range(2))

    @pl.when(pl.program_id(0) == 0)
    def _():
        p2[...] = pp2

    @pl.when(pl.program_id(0) != 0)
    def _():
        p2[...] += pp2


def _mlp2(h1pv, agg1v, oh2, W2abd, b2abd, W2bbd, b2bbd):
    grid = (N // RB,)
    return pl.pallas_call(
        _mlp2_body,
        grid=grid,
        in_specs=[
            pl.BlockSpec((RB // 2, 128), lambda i: (i, 0)),
            pl.BlockSpec((RB // 2, 128), lambda i: (i, 0)),
            pl.BlockSpec((1, RB // 2, 32), lambda i: (i, 0, 0)),
            pl.BlockSpec((128, 128), lambda i: (0, 0)),
            pl.BlockSpec((1, 128), lambda i: (0, 0)),
            pl.BlockSpec((128, 128), lambda i: (0, 0)),
            pl.BlockSpec((1, 128), lambda i: (0, 0)),
        ],
        out_specs=pl.BlockSpec((G, 64), lambda i: (0, 0)),
        out_shape=jax.ShapeDtypeStruct((G, 64), jnp.float32),
    )(h1pv, agg1v, oh2, W2abd, b2abd, W2bbd, b2bbd)


def _heads_body(p0, p1, p2, Wp0, bp0, Wp1, bp1, Wp2, bp2,
                Wm1, bm1, Wm2, bm2, Wmean, bmean, Wstd, bstd, mean, std):
    score = (jnp.dot(p0[...], Wp0[...], preferred_element_type=jnp.float32)
             + bp0[...]
             + jnp.dot(p1[...], Wp1[...], preferred_element_type=jnp.float32)
             + bp1[...]
             + jnp.dot(p2[...], Wp2[...], preferred_element_type=jnp.float32)
             + bp2[...])
    f = jnp.maximum(jnp.dot(score, Wm1[...], preferred_element_type=jnp.float32)
                    + bm1[...], 0.0)
    f = jnp.maximum(jnp.dot(f, Wm2[...], preferred_element_type=jnp.float32)
                    + bm2[...], 0.0)
    mean[...] = jnp.dot(f, Wmean[...], preferred_element_type=jnp.float32) \
        + bmean[...]
    z = jnp.dot(f, Wstd[...], preferred_element_type=jnp.float32) + bstd[...]
    std[...] = jnp.maximum(z, 0.0) + jnp.log1p(jnp.exp(-jnp.abs(z)))


def _heads(p0, p1, p2, Wp0, bp0, Wp1, bp1, Wp2, bp2,
           Wm1, bm1, Wm2, bm2, Wmean, bmean, Wstd, bstd):
    return pl.pallas_call(
        _heads_body,
        out_shape=[
            jax.ShapeDtypeStruct((G, 32), jnp.float32),
            jax.ShapeDtypeStruct((G, 32), jnp.float32),
        ],
    )(p0, p1, p2, Wp0, bp0, Wp1, bp1, Wp2, bp2,
      Wm1, bm1, Wm2, bm2, Wmean, bmean, Wstd, bstd)


def kernel(node_feats, edge_index, graph_ids,
           W1a, b1a, W1b, b1b, W2a, b2a, W2b, b2b,
           Wp0, bp0, Wp1, bp1, Wp2, bp2,
           Wm1, bm1, Wm2, bm2, Wmean, bmean, Wstd, bstd):
    npad = EP - E
    srcp = jnp.concatenate(
        [edge_index[0], jnp.arange(npad, dtype=jnp.int32) % N])
    dst = jnp.concatenate(
        [edge_index[1],
         N + (jnp.arange(npad, dtype=jnp.int32) % (NP - N))]) \
        .reshape(EROWS, 128)
    zeros = jnp.zeros((NP // 8, 128), jnp.float32).reshape(NP, 16)
    oh = (graph_ids[:, None] == jnp.arange(G, dtype=jnp.int32)[None, :]) \
        .astype(jnp.float32)
    oh4 = oh.reshape(N // RB, RB // 4, 64)
    oh2 = oh.reshape(N // RB, RB // 2, 32)

    t0 = node_feats.reshape(8 * N, 16)
    i0 = [(8 * srcp + k).reshape(EROWS, 128) for k in range(2)]
    agg0 = _agg(2, 8 * N)(t0, i0[0], i0[1], dst, zeros)
    agg0v = agg0.reshape(NP // 4, 128)

    h0p = node_feats[:, 0:32].reshape(N // 4, 128)
    bd = jax.scipy.linalg.block_diag
    h1p, p0, p1 = _mlp1(
        h0p, agg0v, oh4,
        bd(W1a, W1a, W1a, W1a), jnp.tile(b1a, 4).reshape(1, 256),
        bd(W1b, W1b, W1b, W1b), jnp.tile(b1b, 4).reshape(1, 256))

    t1 = h1p.reshape(4 * N, 16)
    i1 = [(4 * srcp + k).reshape(EROWS, 128) for k in range(4)]
    agg1 = _agg(4, 4 * N)(t1, i1[0], i1[1], i1[2], i1[3], dst, zeros)
    agg1v = agg1.reshape(NP // 2, 128)

    h1pv = h1p.reshape(N // 2, 128)
    p2 = _mlp2(h1pv, agg1v, oh2,
               bd(W2a, W2a), jnp.tile(b2a, 2).reshape(1, 128),
               bd(W2b, W2b), jnp.tile(b2b, 2).reshape(1, 128))
    mean, std = _heads(
        p0, p1, p2, Wp0, bp0.reshape(1, -1), Wp1, bp1.reshape(1, -1),
        Wp2, bp2.reshape(1, -1), Wm1, bm1.reshape(1, -1),
        Wm2, bm2.reshape(1, -1), Wmean, bmean.reshape(1, -1),
        Wstd, bstd.reshape(1, -1))
    return mean, std


# speedup vs baseline: 16.3162x; 1.1219x over previous
import functools

import jax
import jax.numpy as jnp
from jax import lax
from jax.experimental import pallas as pl
from jax.experimental.pallas import tpu as pltpu
from jax.experimental.pallas import tpu_sc as plsc

N = 100000
E = 1600000
G = 16
NP = 100352
NT = NP // 16
EP = 1638400
EROWS = EP // 128
RT = EROWS // 16
NSB = 20
SBROWS = RT // NSB
NBUF = 4
CS = 2
TRIPS = SBROWS // (NBUF * CS)


def _make_agg(num_groups, table_rows):
    npasses = num_groups // 2
    mesh = plsc.VectorSubcoreMesh(core_axis_name="c", subcore_axis_name="s")

    @functools.partial(
        pl.kernel,
        out_type=jax.ShapeDtypeStruct((NP, num_groups, 16), jnp.float32),
        mesh=mesh,
        scratch_types=(
            [pltpu.VMEM((SBROWS, 128), jnp.int32),
             pltpu.VMEM((SBROWS, 128), jnp.int32)]
            + [pltpu.VMEM((CS * 128, 16), jnp.float32)
               for _ in range(NBUF)]
            + [pltpu.VMEM_SHARED((NP, 16), jnp.float32)]
            + [pltpu.SemaphoreType.DMA for _ in range(2 * NBUF)]
        ),
        compiler_params=pltpu.CompilerParams(use_tc_tiling_on_sc=False),
    )
    def agg(*refs):
        table = refs[0]
        idxs = refs[1:1 + num_groups]
        dstg, zeros_hbm = refs[1 + num_groups:3 + num_groups]
        out = refs[3 + num_groups]
        isrc, idst = refs[4 + num_groups:6 + num_groups]
        rows = refs[6 + num_groups:6 + num_groups + NBUF]
        acc = refs[6 + num_groups + NBUF]
        gsems = refs[7 + num_groups + NBUF:7 + num_groups + 2 * NBUF]
        ssems = refs[7 + num_groups + 2 * NBUF:7 + num_groups + 3 * NBUF]

        c = lax.axis_index("c")
        s = lax.axis_index("s")
        lo = s * NT

        def drain(b):
            pltpu.make_async_copy(
                zeros_hbm.at[pl.ds(0, CS * 128)], rows[b], ssems[b]).wait()

        def one_pass(group):
            idxg = idxs[group]
            pltpu.sync_copy(zeros_hbm.at[pl.ds(lo, NT)], acc.at[pl.ds(lo, NT)])
            plsc.subcore_barrier()

            def superblock(sb, carry2):
                @pl.when(sb > 0)
                def _():
                    for b in range(NBUF):
                        drain(b)
                base_row = s * RT + sb * SBROWS
                pltpu.sync_copy(idxg.at[pl.ds(base_row, SBROWS)], isrc)
                pltpu.sync_copy(dstg.at[pl.ds(base_row, SBROWS)], idst)

                def trip(j, carry):
                    r = j * NBUF * CS
                    hs = []
                    for b in range(NBUF):
                        @pl.when(j > 0)
                        def _(b=b):
                            drain(b)
                        hs.append([
                            pltpu.async_copy(
                                table.at[isrc.at[r + b * CS + k]],
                                rows[b].at[pl.ds(k * 128, 128)], gsems[b])
                            for k in range(CS)
                        ])
                    for b in range(NBUF):
                        for h in hs[b]:
                            h.wait()
                        for k in range(CS):
                            pltpu.async_copy(
                                rows[b].at[pl.ds(k * 128, 128)],
                                acc.at[idst.at[r + b * CS + k]],
                                ssems[b], add=True)
                    return carry

                lax.fori_loop(0, TRIPS, trip, 0)
                return carry2

            lax.fori_loop(0, NSB, superblock, 0)
            for b in range(NBUF):
                drain(b)
            plsc.subcore_barrier()
            pltpu.sync_copy(acc.at[pl.ds(lo, NT)],
                            out.at[pl.ds(lo, NT), group])

        def run(groups):
            for g in groups:
                one_pass(g)

        pl.when(c == 0)(lambda: run(range(npasses)))
        pl.when(c == 1)(lambda: run(range(npasses, num_groups)))

    return agg


@functools.cache
def _agg(num_groups, table_rows):
    return _make_agg(num_groups, table_rows)


RB = 4000


def _mlp1_body(h0p, agg0, oh4, W1abd, b1abd, W1bbd, b1bbd, h1p, p0, p1):
    h0 = h0p[...]
    x = h0 + agg0[...]
    t = jnp.maximum(jnp.dot(x, W1abd[...], preferred_element_type=jnp.float32)
                    + b1abd[...], 0.0)
    h1 = jnp.maximum(jnp.dot(t, W1bbd[...], preferred_element_type=jnp.float32)
                     + b1bbd[...], 0.0)
    h1p[:, 0:1, :] = h1[:, 0:128].reshape(RB // 4, 1, 128)
    h1p[:, 1:2, :] = h1[:, 128:256].reshape(RB // 4, 1, 128)
    ohb = oh4[0]
    m0 = lax.dot_general(ohb, h0, (((0,), (0,)), ((), ())),
                         preferred_element_type=jnp.float32)
    m1 = lax.dot_general(ohb, h1, (((0,), (0,)), ((), ())),
                         preferred_element_type=jnp.float32)
    pp0 = sum(m0[16 * j:16 * (j + 1), 32 * j:32 * (j + 1)] for j in range(4))
    pp1 = sum(m1[16 * j:16 * (j + 1), 64 * j:64 * (j + 1)] for j in range(4))

    @pl.when(pl.program_id(0) == 0)
    def _():
        p0[...] = pp0
        p1[...] = pp1

    @pl.when(pl.program_id(0) != 0)
    def _():
        p0[...] += pp0
        p1[...] += pp1


def _mlp1(h0p, agg0v, oh4, W1abd, b1abd, W1bbd, b1bbd):
    grid = (N // RB,)
    return pl.pallas_call(
        _mlp1_body,
        grid=grid,
        in_specs=[
            pl.BlockSpec((RB // 4, 128), lambda i: (i, 0)),
            pl.BlockSpec((RB // 4, 128), lambda i: (i, 0)),
            pl.BlockSpec((1, RB // 4, 64), lambda i: (i, 0, 0)),
            pl.BlockSpec((128, 256), lambda i: (0, 0)),
            pl.BlockSpec((1, 256), lambda i: (0, 0)),
            pl.BlockSpec((256, 256), lambda i: (0, 0)),
            pl.BlockSpec((1, 256), lambda i: (0, 0)),
        ],
        out_specs=[
            pl.BlockSpec((RB // 4, 2, 128), lambda i: (i, 0, 0)),
            pl.BlockSpec((G, 32), lambda i: (0, 0)),
            pl.BlockSpec((G, 64), lambda i: (0, 0)),
        ],
        out_shape=[
            jax.ShapeDtypeStruct((N // 4, 2, 128), jnp.float32),
            jax.ShapeDtypeStruct((G, 32), jnp.float32),
            jax.ShapeDtypeStruct((G, 64), jnp.float32),
        ],
    )(h0p, agg0v, oh4, W1abd, b1abd, W1bbd, b1bbd)


def _mlp2_body(h1pv, agg1, oh2, W2abd, b2abd, W2bbd, b2bbd, p2):
    x = h1pv[...] + agg1[...]
    t = jnp.maximum(jnp.dot(x, W2abd[...], preferred_element_type=jnp.float32)
                    + b2abd[...], 0.0)
    h2 = jnp.maximum(jnp.dot(t, W2bbd[...], preferred_element_type=jnp.float32)
                     + b2bbd[...], 0.0)
    m2 = lax.dot_general(oh2[0], h2, (((0,), (0,)), ((), ())),
                         preferred_element_type=jnp.float32)
    pp2 = sum(m2[16 * j:16 * (j + 1), 64 * j:64 * (j + 1)] for j in range(2))

    @pl.when(pl.program_id(0) == 0)
    def _():
        p2[...] = pp2

    @pl.when(pl.program_id(0) != 0)
    def _():
        p2[...] += pp2


def _mlp2(h1pv, agg1v, oh2, W2abd, b2abd, W2bbd, b2bbd):
    grid = (N // RB,)
    return pl.pallas_call(
        _mlp2_body,
        grid=grid,
        in_specs=[
            pl.BlockSpec((RB // 2, 128), lambda i: (i, 0)),
            pl.BlockSpec((RB // 2, 128), lambda i: (i, 0)),
            pl.BlockSpec((1, RB // 2, 32), lambda i: (i, 0, 0)),
            pl.BlockSpec((128, 128), lambda i: (0, 0)),
            pl.BlockSpec((1, 128), lambda i: (0, 0)),
            pl.BlockSpec((128, 128), lambda i: (0, 0)),
            pl.BlockSpec((1, 128), lambda i: (0, 0)),
        ],
        out_specs=pl.BlockSpec((G, 64), lambda i: (0, 0)),
        out_shape=jax.ShapeDtypeStruct((G, 64), jnp.float32),
    )(h1pv, agg1v, oh2, W2abd, b2abd, W2bbd, b2bbd)


def _heads_body(p0, p1, p2, Wp0, bp0, Wp1, bp1, Wp2, bp2,
                Wm1, bm1, Wm2, bm2, Wmean, bmean, Wstd, bstd, mean, std):
    score = (jnp.dot(p0[...], Wp0[...], preferred_element_type=jnp.float32)
             + bp0[...]
             + jnp.dot(p1[...], Wp1[...], preferred_element_type=jnp.float32)
             + bp1[...]
             + jnp.dot(p2[...], Wp2[...], preferred_element_type=jnp.float32)
             + bp2[...])
    f = jnp.maximum(jnp.dot(score, Wm1[...], preferred_element_type=jnp.float32)
                    + bm1[...], 0.0)
    f = jnp.maximum(jnp.dot(f, Wm2[...], preferred_element_type=jnp.float32)
                    + bm2[...], 0.0)
    mean[...] = jnp.dot(f, Wmean[...], preferred_element_type=jnp.float32) \
        + bmean[...]
    z = jnp.dot(f, Wstd[...], preferred_element_type=jnp.float32) + bstd[...]
    std[...] = jnp.maximum(z, 0.0) + jnp.log1p(jnp.exp(-jnp.abs(z)))


def _heads(p0, p1, p2, Wp0, bp0, Wp1, bp1, Wp2, bp2,
           Wm1, bm1, Wm2, bm2, Wmean, bmean, Wstd, bstd):
    return pl.pallas_call(
        _heads_body,
        out_shape=[
            jax.ShapeDtypeStruct((G, 32), jnp.float32),
            jax.ShapeDtypeStruct((G, 32), jnp.float32),
        ],
    )(p0, p1, p2, Wp0, bp0, Wp1, bp1, Wp2, bp2,
      Wm1, bm1, Wm2, bm2, Wmean, bmean, Wstd, bstd)


def kernel(node_feats, edge_index, graph_ids,
           W1a, b1a, W1b, b1b, W2a, b2a, W2b, b2b,
           Wp0, bp0, Wp1, bp1, Wp2, bp2,
           Wm1, bm1, Wm2, bm2, Wmean, bmean, Wstd, bstd):
    npad = EP - E
    srcp = jnp.concatenate(
        [edge_index[0], jnp.arange(npad, dtype=jnp.int32) % N])
    dst = jnp.concatenate(
        [edge_index[1],
         N + (jnp.arange(npad, dtype=jnp.int32) % (NP - N))]) \
        .reshape(EROWS, 128)
    zeros = jnp.zeros((NP // 8, 128), jnp.float32).reshape(NP, 16)
    oh = (graph_ids[:, None] == jnp.arange(G, dtype=jnp.int32)[None, :]) \
        .astype(jnp.float32)
    oh4 = oh.reshape(N // RB, RB // 4, 64)
    oh2 = oh.reshape(N // RB, RB // 2, 32)

    t0 = node_feats.reshape(8 * N, 16)
    i0 = [(8 * srcp + k).reshape(EROWS, 128) for k in range(2)]
    agg0 = _agg(2, 8 * N)(t0, i0[0], i0[1], dst, zeros)
    agg0v = agg0.reshape(NP // 4, 128)

    h0p = node_feats[:, 0:32].reshape(N // 4, 128)
    bd = jax.scipy.linalg.block_diag
    h1p, p0, p1 = _mlp1(
        h0p, agg0v, oh4,
        bd(W1a, W1a, W1a, W1a), jnp.tile(b1a, 4).reshape(1, 256),
        bd(W1b, W1b, W1b, W1b), jnp.tile(b1b, 4).reshape(1, 256))

    t1 = h1p.reshape(4 * N, 16)
    i1 = [(4 * srcp + k).reshape(EROWS, 128) for k in range(4)]
    agg1 = _agg(4, 4 * N)(t1, i1[0], i1[1], i1[2], i1[3], dst, zeros)
    agg1v = agg1.reshape(NP // 2, 128)

    h1pv = h1p.reshape(N // 2, 128)
    p2 = _mlp2(h1pv, agg1v, oh2,
               bd(W2a, W2a), jnp.tile(b2a, 2).reshape(1, 128),
               bd(W2b, W2b), jnp.tile(b2b, 2).reshape(1, 128))
    mean, std = _heads(
        p0, p1, p2, Wp0, bp0.reshape(1, -1), Wp1, bp1.reshape(1, -1),
        Wp2, bp2.reshape(1, -1), Wm1, bm1.reshape(1, -1),
        Wm2, bm2.reshape(1, -1), Wmean, bmean.reshape(1, -1),
        Wstd, bstd.reshape(1, -1))
    return mean, std
